# baseline (speedup 1.0000x reference)
import functools

import jax
import jax.numpy as jnp
from jax import lax
from jax.experimental import pallas as pl
from jax.experimental.pallas import tpu as pltpu
from jax.experimental.pallas import tpu_sc as plsc

_ACQ = 1
_B, _S, _D = 1024, 4, 2048
_NC, _NS = 2, 16
_NW = _NC * _NS
_RPW = _B // _NW


_RPC = _B // _NC


@functools.partial(
    pl.kernel,
    mesh=plsc.ScalarSubcoreMesh(axis_name="c", num_cores=_NC),
    out_type=jax.ShapeDtypeStruct((_B, 1, _D), jnp.float32),
    scratch_types=[pltpu.VMEM_SHARED((_RPC, 1, _D), jnp.float32)],
)
def _split(in_hbm, out_hbm, buf_s):
    cid = lax.axis_index("c")
    base = cid * _RPC
    pltpu.sync_copy(in_hbm.at[pl.ds(base, _RPC), pl.ds(_ACQ, 1), :], buf_s)
    pltpu.sync_copy(buf_s, out_hbm.at[pl.ds(base, _RPC)])


def kernel(inputs):
    return _split(inputs).reshape(_B, _D)


# speedup vs baseline: 1.0776x; 1.0776x over previous
import functools

import jax
import jax.numpy as jnp
from jax import lax
from jax.experimental import pallas as pl
from jax.experimental.pallas import tpu as pltpu
from jax.experimental.pallas import tpu_sc as plsc

_ACQ = 1
_B, _S, _D = 1024, 4, 2048
_NC, _NS = 2, 16
_NW = _NC * _NS
_RPW = _B // _NW
_NCH = 4
_CH = _RPW // _NCH


@functools.partial(
    pl.kernel,
    mesh=plsc.VectorSubcoreMesh(core_axis_name="c", subcore_axis_name="s"),
    out_type=jax.ShapeDtypeStruct((_B, 1, _D), jnp.float32),
    scratch_types=[
        pltpu.VMEM((_CH, 1, _D), jnp.float32),
        pltpu.VMEM((_CH, 1, _D), jnp.float32),
        pltpu.SemaphoreType.DMA,
        pltpu.SemaphoreType.DMA,
        pltpu.SemaphoreType.DMA,
        pltpu.SemaphoreType.DMA,
    ],
)
def _split(in_hbm, out_hbm, b0, b1, sg0, sg1, ss0, ss1):
    wid = lax.axis_index("s") * _NC + lax.axis_index("c")
    base = wid * _RPW
    bufs, gsems, ssems = (b0, b1), (sg0, sg1), (ss0, ss1)

    def gcopy(k):
        return pltpu.make_async_copy(
            in_hbm.at[pl.ds(base + k * _CH, _CH), pl.ds(_ACQ, 1), :],
            bufs[k % 2],
            gsems[k % 2],
        )

    def scopy(k):
        return pltpu.make_async_copy(
            bufs[k % 2],
            out_hbm.at[pl.ds(base + k * _CH, _CH)],
            ssems[k % 2],
        )

    gcopy(0).start()
    for k in range(_NCH):
        if k + 1 < _NCH:
            if k >= 1:
                scopy(k - 1).wait()
            gcopy(k + 1).start()
        gcopy(k).wait()
        scopy(k).start()
    scopy(_NCH - 2).wait()
    scopy(_NCH - 1).wait()


def kernel(inputs):
    return _split(inputs).reshape(_B, _D)
